# baseline (speedup 1.0000x reference)
import jax
import jax.numpy as jnp
from jax.experimental import pallas as pl


def _relu_bias_kernel(x_ref, b_ref, o_ref):
    o_ref[...] = jnp.maximum(x_ref[...] + b_ref[...], 0.0)


def _relu_bias(x, b):
    n, d = x.shape
    return pl.pallas_call(
        _relu_bias_kernel,
        out_shape=jax.ShapeDtypeStruct((n, d), x.dtype),
    )(x, jnp.broadcast_to(b, (n, d)))


def _layer(H, hyperedges, rv, W, b):
    n = H.shape[0]
    k = hyperedges.shape[1]
    proj = H @ rv
    p = proj[hyperedges]
    s = jnp.argmax(p, axis=1)
    i = jnp.argmin(p, axis=1)
    Se = jnp.take_along_axis(hyperedges, s[:, None], axis=1)[:, 0]
    Ie = jnp.take_along_axis(hyperedges, i[:, None], axis=1)[:, 0]
    rows = jnp.concatenate([Se, Ie])
    cols = jnp.concatenate([Ie, Se])
    vals = jnp.full(rows.shape, 1.0 / k, dtype=H.dtype)
    deg = jnp.zeros((n,), dtype=H.dtype).at[rows].add(vals) + 1.0
    dinv = deg ** -0.5
    HW = H @ W
    coef = vals * dinv[rows] * dinv[cols]
    out = jnp.zeros_like(HW).at[rows].add(coef[:, None] * HW[cols])
    out = out + (dinv * dinv)[:, None] * HW
    return _relu_bias(out, b)


def kernel(x, hyperedges, rv1, rv2, W1, b1, W2, b2):
    H = _layer(x, hyperedges, rv1, W1, b1)
    H = _layer(H, hyperedges, rv2, W2, b2)
    return H


# speedup vs baseline: 202.6272x; 202.6272x over previous
import functools

import jax
import jax.numpy as jnp
from jax import lax
from jax.experimental import pallas as pl
from jax.experimental.pallas import tpu as pltpu
from jax.experimental.pallas import tpu_sc as plsc

N = 10000
NPAD = 10240
NE = 320000
K = 8
NC = 2
NS = 16
NW = NC * NS
EPT = NE // NW
CHUNK = 2000
NCHUNKS = EPT // CHUNK
SLICE = NPAD // NS
F = 16

_mesh = plsc.VectorSubcoreMesh(core_axis_name="c", subcore_axis_name="s")


def _tc_dense(x_pad, rv, W):
    din = x_pad.shape[1]

    def kfn(x_ref, rv_ref, w_ref, p_ref, hw_ref):
        xb = x_ref[...]
        p_ref[...] = jnp.dot(xb, rv_ref[...], preferred_element_type=jnp.float32)
        hw_ref[...] = jnp.dot(xb, w_ref[...], preferred_element_type=jnp.float32)

    return pl.pallas_call(
        kfn,
        grid=(NPAD // 1024,),
        in_specs=[
            pl.BlockSpec((1024, din), lambda i: (i, 0)),
            pl.BlockSpec((din, 1), lambda i: (0, 0)),
            pl.BlockSpec((din, F), lambda i: (0, 0)),
        ],
        out_specs=[
            pl.BlockSpec((1024, 1), lambda i: (i, 0)),
            pl.BlockSpec((1024, F), lambda i: (i, 0)),
        ],
        out_shape=[
            jax.ShapeDtypeStruct((NPAD, 1), jnp.float32),
            jax.ShapeDtypeStruct((NPAD, F), jnp.float32),
        ],
    )(x_pad, rv.reshape(din, 1), W)


@functools.partial(
    pl.kernel,
    out_type=[
        jax.ShapeDtypeStruct((NE,), jnp.int32),
        jax.ShapeDtypeStruct((NE,), jnp.int32),
        jax.ShapeDtypeStruct((NC, NS, SLICE), jnp.float32),
    ],
    mesh=_mesh,
    compiler_params=pltpu.CompilerParams(needs_layout_passes=False, use_tc_tiling_on_sc=False),
    scratch_types=[
        pltpu.VMEM((NPAD,), jnp.float32),
        pltpu.VMEM((CHUNK * K,), jnp.int32),
        pltpu.VMEM((CHUNK,), jnp.int32),
        pltpu.VMEM((CHUNK,), jnp.int32),
        pltpu.VMEM((CHUNK,), jnp.float32),
        pltpu.VMEM((SLICE,), jnp.float32),
        pltpu.VMEM_SHARED((NPAD,), jnp.float32),
    ],
)
def _sc_edges(proj_hbm, edges_hbm, se_hbm, ie_hbm, deg_hbm,
              proj_v, ebuf, sebuf, iebuf, valbuf, stage, degsh):
    cid = lax.axis_index("c")
    sid = lax.axis_index("s")
    ebase = (cid * NS + sid) * EPT

    pltpu.sync_copy(proj_hbm, proj_v)

    def init_val(i, _):
        valbuf[pl.ds(i * 16, 16)] = jnp.full((16,), 0.125, jnp.float32)
        return 0

    lax.fori_loop(0, CHUNK // 16, init_val, 0)

    def init_zero(i, _):
        stage[pl.ds(i * 16, 16)] = jnp.zeros((16,), jnp.float32)
        return 0

    lax.fori_loop(0, SLICE // 16, init_zero, 0)
    pltpu.sync_copy(stage, degsh.at[pl.ds(sid * SLICE, SLICE)])
    plsc.subcore_barrier()

    lanes = lax.iota(jnp.int32, 16)

    def chunk_body(c, _):
        pltpu.sync_copy(
            edges_hbm.at[pl.ds((ebase + c * CHUNK) * K, CHUNK * K)], ebuf)

        def group_body(g, _):
            idx0 = g * (16 * K) + lanes * K
            n_cur = plsc.load_gather(ebuf, [idx0])
            p_cur = plsc.load_gather(proj_v, [n_cur])
            nmax = n_cur
            pmax = p_cur
            nmin = n_cur
            pmin = p_cur
            for j in range(1, K):
                nj = plsc.load_gather(ebuf, [idx0 + j])
                pj = plsc.load_gather(proj_v, [nj])
                gt = pj > pmax
                nmax = jnp.where(gt, nj, nmax)
                pmax = jnp.where(gt, pj, pmax)
                ltm = pj < pmin
                nmin = jnp.where(ltm, nj, nmin)
                pmin = jnp.where(ltm, pj, pmin)
            sebuf[pl.ds(g * 16, 16)] = nmax
            iebuf[pl.ds(g * 16, 16)] = nmin
            return 0

        lax.fori_loop(0, CHUNK // 16, group_body, 0)
        pltpu.sync_copy(sebuf, se_hbm.at[pl.ds(ebase + c * CHUNK, CHUNK)])
        pltpu.sync_copy(iebuf, ie_hbm.at[pl.ds(ebase + c * CHUNK, CHUNK)])
        pltpu.sync_copy(valbuf, degsh.at[sebuf], add=True)
        pltpu.sync_copy(valbuf, degsh.at[iebuf], add=True)
        return 0

    lax.fori_loop(0, NCHUNKS, chunk_body, 0)

    plsc.subcore_barrier()
    pltpu.sync_copy(degsh.at[pl.ds(sid * SLICE, SLICE)], stage)
    pltpu.sync_copy(stage, deg_hbm.at[cid, sid])


def _tc_norm(degp, hw):
    def kfn(d_ref, hw_ref, dinv_ref, gs_ref):
        deg = d_ref[0] + d_ref[1] + 1.0
        dinv = lax.rsqrt(deg)
        dinv_ref[...] = dinv
        gs_ref[...] = dinv * hw_ref[...] * 0.125

    return pl.pallas_call(
        kfn,
        grid=(NPAD // 1024,),
        in_specs=[
            pl.BlockSpec((2, 1024, 1), lambda i: (0, i, 0)),
            pl.BlockSpec((1024, F), lambda i: (i, 0)),
        ],
        out_specs=[
            pl.BlockSpec((1024, 1), lambda i: (i, 0)),
            pl.BlockSpec((1024, F), lambda i: (i, 0)),
        ],
        out_shape=[
            jax.ShapeDtypeStruct((NPAD, 1), jnp.float32),
            jax.ShapeDtypeStruct((NPAD, F), jnp.float32),
        ],
    )(degp, hw)


@functools.partial(
    pl.kernel,
    out_type=jax.ShapeDtypeStruct((NC, NS, SLICE, F), jnp.float32),
    mesh=_mesh,
    compiler_params=pltpu.CompilerParams(needs_layout_passes=False, use_tc_tiling_on_sc=False),
    scratch_types=[
        pltpu.VMEM((CHUNK,), jnp.int32),
        pltpu.VMEM((CHUNK,), jnp.int32),
        pltpu.VMEM((CHUNK, F), jnp.float32),
        pltpu.VMEM((SLICE, F), jnp.float32),
        pltpu.VMEM_SHARED((NPAD, F), jnp.float32),
        pltpu.VMEM_SHARED((NPAD, F), jnp.float32),
    ],
)
def _sc_msgs(se_hbm, ie_hbm, gs_hbm, acc_hbm,
             sebuf, iebuf, rbuf, stage, gssh, accsh):
    cid = lax.axis_index("c")
    sid = lax.axis_index("s")
    ebase = (cid * NS + sid) * EPT
    rbase = sid * SLICE

    pltpu.sync_copy(gs_hbm.at[pl.ds(rbase, SLICE)], stage)
    pltpu.sync_copy(stage, gssh.at[pl.ds(rbase, SLICE)])

    def init_zero(i, _):
        stage[i, :] = jnp.zeros((F,), jnp.float32)
        return 0

    lax.fori_loop(0, SLICE, init_zero, 0)
    pltpu.sync_copy(stage, accsh.at[pl.ds(rbase, SLICE)])
    plsc.subcore_barrier()

    def chunk_body(c, _):
        pltpu.sync_copy(se_hbm.at[pl.ds(ebase + c * CHUNK, CHUNK)], sebuf)
        pltpu.sync_copy(ie_hbm.at[pl.ds(ebase + c * CHUNK, CHUNK)], iebuf)
        pltpu.sync_copy(gssh.at[iebuf], rbuf)
        pltpu.sync_copy(rbuf, accsh.at[sebuf], add=True)
        pltpu.sync_copy(gssh.at[sebuf], rbuf)
        pltpu.sync_copy(rbuf, accsh.at[iebuf], add=True)
        return 0

    lax.fori_loop(0, NCHUNKS, chunk_body, 0)

    plsc.subcore_barrier()
    pltpu.sync_copy(accsh.at[pl.ds(rbase, SLICE)], stage)
    pltpu.sync_copy(stage, acc_hbm.at[cid, sid])


def _tc_combine(accp, dinv, gs, b):
    def kfn(a_ref, dinv_ref, gs_ref, b_ref, o_ref):
        acc = a_ref[0] + a_ref[1]
        o_ref[...] = jnp.maximum(
            dinv_ref[...] * (acc + 8.0 * gs_ref[...]) + b_ref[...], 0.0)

    return pl.pallas_call(
        kfn,
        grid=(NPAD // 1024,),
        in_specs=[
            pl.BlockSpec((2, 1024, F), lambda i: (0, i, 0)),
            pl.BlockSpec((1024, 1), lambda i: (i, 0)),
            pl.BlockSpec((1024, F), lambda i: (i, 0)),
            pl.BlockSpec((1, F), lambda i: (0, 0)),
        ],
        out_specs=pl.BlockSpec((1024, F), lambda i: (i, 0)),
        out_shape=jax.ShapeDtypeStruct((NPAD, F), jnp.float32),
    )(accp, dinv, gs, b)


def _layer(H_pad, edges_flat, rv, W, b):
    proj, hw = _tc_dense(H_pad, rv, W)
    se, ie, degp = _sc_edges(proj.reshape(NPAD), edges_flat)
    dinv, gs = _tc_norm(degp.reshape(NC, NPAD, 1), hw)
    accp = _sc_msgs(se, ie, gs)
    return _tc_combine(accp.reshape(NC, NPAD, F), dinv, gs, b.reshape(1, F))


def kernel(x, hyperedges, rv1, rv2, W1, b1, W2, b2):
    x_pad = jnp.concatenate(
        [x, jnp.zeros((NPAD - N, x.shape[1]), x.dtype)], axis=0)
    edges_flat = hyperedges.reshape(-1)
    H = _layer(x_pad, edges_flat, rv1, W1, b1)
    H = _layer(H, edges_flat, rv2, W2, b2)
    return H[:N]
